# Initial kernel scaffold; baseline (speedup 1.0000x reference)
import jax
import jax.numpy as jnp
from jax.experimental import pallas as pl


def kernel(x, indices):
    raise NotImplementedError("write your pallas kernel here")


# speedup vs baseline: 9.7109x; 9.7109x over previous
import functools

import jax
import jax.numpy as jnp
from jax import lax
from jax.experimental import pallas as pl
from jax.experimental.pallas import tpu as pltpu
from jax.experimental.pallas import tpu_sc as plsc

B, N, C = 4, 786432, 16
NC, NS = 2, 16
NW = NC * NS
ROWS_PER_W = B * N // NW
W_PER_B = NW // B
CHUNK = 2048
GSZ = 128
NGATH = CHUNK // GSZ
N_CHUNKS = ROWS_PER_W // CHUNK


@functools.partial(
    pl.kernel,
    out_type=jax.ShapeDtypeStruct((B, N, C), jnp.float32),
    mesh=plsc.VectorSubcoreMesh(core_axis_name="c", subcore_axis_name="s"),
    compiler_params=pltpu.CompilerParams(use_tc_tiling_on_sc=False),
    scratch_types=[
        pltpu.VMEM((2, CHUNK), jnp.int32),
        pltpu.VMEM((2, NGATH, GSZ), jnp.int32),
        pltpu.VMEM((2, CHUNK, C), jnp.float32),
        pltpu.VMEM((16,), jnp.int32),
        pltpu.SemaphoreType.DMA,
        pltpu.SemaphoreType.DMA,
        pltpu.SemaphoreType.DMA,
    ],
)
def _order_map_sc(x_hbm, idx_hbm, out_hbm, idxraw, gidx, rows, padacc,
                  gsem, osem0, osem1):
    wid = lax.axis_index("s") * NC + lax.axis_index("c")
    b = wid // W_PER_B
    ibase = (wid % W_PER_B) * ROWS_PER_W
    xb = x_hbm.at[b]
    ob = out_hbm.at[b]
    osems = (osem0, osem1)

    def do_chunk(c, k):
        idx_k = idxraw.at[k]
        gidx_k = gidx.at[k]
        rows_k = rows.at[k]
        pos = ibase + c * CHUNK
        out_cp = pltpu.make_async_copy(
            rows_k, ob.at[pl.ds(pos, CHUNK)], osems[k])

        @pl.when(c >= 2)
        def _drain():
            pltpu.make_async_copy(
                rows_k, ob.at[pl.ds(pos - 2 * CHUNK, CHUNK)], osems[k]).wait()

        pltpu.sync_copy(idx_hbm.at[pl.ds(pos, CHUNK)], idx_k)

        padacc[...] = jnp.zeros((16,), jnp.int32)

        def tbody(j, _):
            for t in range(GSZ // 16):
                v = idx_k[pl.ds(j * GSZ + t * 16, 16)]
                padacc[...] = padacc[...] + jnp.where(v == N, 1, 0)
                gidx_k[j, pl.ds(t * 16, 16)] = jnp.minimum(v, N - 1)
            return 0
        lax.fori_loop(0, NGATH, tbody, 0)
        pv = padacc[...]
        npad = sum(pv[i] for i in range(16))

        copies = [
            pltpu.make_async_copy(
                xb.at[gidx_k.at[j]], rows_k.at[pl.ds(j * GSZ, GSZ)], gsem)
            for j in range(NGATH)
        ]
        for cp in copies:
            cp.start()
        for cp in copies:
            cp.wait()

        @pl.when(npad > 0)
        def _fix():
            def fbody(g, _):
                v = idx_k[pl.ds(g * 16, 16)]
                for r in range(16):
                    scale = jnp.where(v[r] == N, 0.0, 1.0)
                    ridx = g * 16 + r
                    rows_k[ridx, :] = rows_k[ridx, :] * scale
                return 0
            lax.fori_loop(0, CHUNK // 16, fbody, 0)

        out_cp.start()

    def pair_body(co, carry):
        do_chunk(2 * co, 0)
        do_chunk(2 * co + 1, 1)
        return carry

    lax.fori_loop(0, N_CHUNKS // 2, pair_body, 0)

    for k in range(2):
        pos = ibase + (N_CHUNKS - 2 + k) * CHUNK
        pltpu.make_async_copy(
            rows.at[k], ob.at[pl.ds(pos, CHUNK)], osems[k]).wait()


def kernel(x, indices):
    return _order_map_sc(x, indices)
